# baseline (speedup 1.0000x reference)
import functools

import jax
import jax.numpy as jnp
from jax import lax
from jax.experimental import pallas as pl
from jax.experimental.pallas import tpu as pltpu
from jax.experimental.pallas import tpu_sc as plsc

D_MODEL = 768
D_FFN = 3072
N_EXPERTS = 8
T = 2048
F_BLK = 768
N_FBLK = D_FFN // F_BLK
ROWS = 256
MAX_TILES = 24
P = MAX_TILES * ROWS
NW = 32
TPW = T // NW


def _router_body(x_ref, wg_ref, pos1_ref, pos2_ref, w1n_ref, w2n_ref,
                 arre_ref, valid_ref, xblk_ref, noff_ref, ncnt_ref, aux_ref):
    xb = x_ref[...]
    logits = jnp.dot(xb, wg_ref[...], preferred_element_type=jnp.float32)
    mu = jnp.mean(logits, axis=1, keepdims=True)
    var = jnp.sum((logits - mu) ** 2, axis=1, keepdims=True) / (N_EXPERTS - 1)
    aux_ref[...] = jnp.mean(var).reshape(1, 1)

    m1 = jnp.max(logits, axis=1, keepdims=True)
    p = jnp.exp(logits - m1)
    probs = p / jnp.sum(p, axis=1, keepdims=True)
    iota8 = lax.broadcasted_iota(jnp.int32, (T, N_EXPERTS), 1)
    w1v = jnp.max(probs, axis=1, keepdims=True)
    e1 = jnp.min(jnp.where(probs == w1v, iota8, N_EXPERTS), axis=1, keepdims=True)
    oh1 = iota8 == e1
    pm = jnp.where(oh1, -1e30, probs)
    w2v = jnp.max(pm, axis=1, keepdims=True)
    e2 = jnp.min(jnp.where(pm == w2v, iota8, N_EXPERTS), axis=1, keepdims=True)
    oh2 = iota8 == e2
    wsum = w1v + w2v
    w1n_ref[...] = w1v / wsum
    w2n_ref[...] = w2v / wsum

    mask = oh1.astype(jnp.float32) + oh2.astype(jnp.float32)
    ir = lax.broadcasted_iota(jnp.int32, (T, T), 0)
    ic = lax.broadcasted_iota(jnp.int32, (T, T), 1)
    lower = (ir > ic).astype(jnp.float32)
    rank = jnp.dot(lower, mask, preferred_element_type=jnp.float32)

    cnt = jnp.sum(mask, axis=0, keepdims=True)
    padded = jnp.ceil(cnt / ROWS) * ROWS
    k8 = lax.broadcasted_iota(jnp.int32, (N_EXPERTS, N_EXPERTS), 0)
    e8 = lax.broadcasted_iota(jnp.int32, (N_EXPERTS, N_EXPERTS), 1)
    lt8 = (k8 < e8).astype(jnp.float32)
    offs = jnp.dot(padded, lt8, preferred_element_type=jnp.float32)
    n_act = jnp.sum(padded) / ROWS

    pos1_ref[...] = (jnp.sum(jnp.where(oh1, rank + offs, 0.0), axis=1,
                             keepdims=True)).astype(jnp.int32)
    pos2_ref[...] = (jnp.sum(jnp.where(oh2, rank + offs, 0.0), axis=1,
                             keepdims=True)).astype(jnp.int32)

    it = lax.broadcasted_iota(jnp.int32, (MAX_TILES, N_EXPERTS), 0).astype(jnp.float32)
    ie = lax.broadcasted_iota(jnp.int32, (MAX_TILES, N_EXPERTS), 1).astype(jnp.float32)
    i_c = jnp.minimum(it, n_act - 1.0)
    start = offs / ROWS
    ntil = padded / ROWS
    ind = ((i_c >= start) & (i_c < start + ntil)).astype(jnp.float32)
    arre_ref[...] = jnp.sum(ie * ind, axis=1, keepdims=True).astype(jnp.int32)
    it1 = lax.broadcasted_iota(jnp.int32, (MAX_TILES, 1), 0).astype(jnp.float32)
    valid_ref[...] = (it1 < n_act).astype(jnp.int32)
    xblk_ref[...] = jnp.minimum(it1, n_act - 1.0).astype(jnp.int32)

    r8 = lax.broadcasted_iota(jnp.int32, (N_EXPERTS, N_EXPERTS), 0)
    c8 = lax.broadcasted_iota(jnp.int32, (N_EXPERTS, N_EXPERTS), 1)
    diag = r8 == c8
    noff_ref[...] = jnp.sum(jnp.where(diag, start, 0.0), axis=1,
                            keepdims=True).astype(jnp.int32)
    ncnt_ref[...] = jnp.sum(jnp.where(diag, ntil, 0.0), axis=1,
                            keepdims=True).astype(jnp.int32)


def _router(x2, Wg):
    outs = pl.pallas_call(
        _router_body,
        out_shape=[
            jax.ShapeDtypeStruct((T, 1), jnp.int32),
            jax.ShapeDtypeStruct((T, 1), jnp.int32),
            jax.ShapeDtypeStruct((T, 1), jnp.float32),
            jax.ShapeDtypeStruct((T, 1), jnp.float32),
            jax.ShapeDtypeStruct((MAX_TILES, 1), jnp.int32),
            jax.ShapeDtypeStruct((MAX_TILES, 1), jnp.int32),
            jax.ShapeDtypeStruct((MAX_TILES, 1), jnp.int32),
            jax.ShapeDtypeStruct((N_EXPERTS, 1), jnp.int32),
            jax.ShapeDtypeStruct((N_EXPERTS, 1), jnp.int32),
            jax.ShapeDtypeStruct((1, 1), jnp.float32),
        ],
    )(x2, Wg)
    return outs


@functools.cache
def _get_dispatch():
    mesh = plsc.VectorSubcoreMesh(core_axis_name="c", subcore_axis_name="s")

    @functools.partial(
        pl.kernel,
        out_type=jax.ShapeDtypeStruct((P, D_MODEL), jnp.float32),
        mesh=mesh,
        scratch_types=[
            pltpu.VMEM((TPW, D_MODEL), jnp.float32),
            pltpu.VMEM((TPW,), jnp.int32),
            pltpu.VMEM((TPW,), jnp.int32),
            pltpu.SemaphoreType.DMA,
        ],
    )
    def _dispatch(x_hbm, pos_hbm, xs_hbm, buf, idx1, idx2, sem):
        wid = lax.axis_index("s") * 2 + lax.axis_index("c")
        base = wid * TPW
        pltpu.sync_copy(x_hbm.at[pl.ds(base, TPW)], buf)
        pltpu.sync_copy(pos_hbm.at[wid], idx1)
        pltpu.sync_copy(pos_hbm.at[NW + wid], idx2)
        pltpu.async_copy(buf, xs_hbm.at[idx1], sem).wait()
        pltpu.async_copy(buf, xs_hbm.at[idx2], sem).wait()

    return _dispatch


def _ffn_body(noff, ncnt, xs_ref, w1_ref, w3_ref, w2_ref, y_ref,
              x_sc, y_sc, sem_in, sem_out):
    e = pl.program_id(0)

    def tile(j, carry):
        row0 = (noff[e] + j) * ROWS
        pltpu.make_async_copy(
            xs_ref.at[pl.ds(row0, ROWS)], x_sc, sem_in).start()
        pltpu.make_async_copy(
            xs_ref.at[pl.ds(row0, ROWS)], x_sc, sem_in).wait()
        xb = x_sc[...]
        a = jnp.dot(xb, w1_ref[0], preferred_element_type=jnp.float32)
        b = jnp.dot(xb, w3_ref[0], preferred_element_type=jnp.float32)
        h = (a * jax.nn.sigmoid(a)) * b
        y_sc[...] = jnp.dot(h, w2_ref[0], preferred_element_type=jnp.float32)
        pltpu.make_async_copy(
            y_sc, y_ref.at[pl.ds(row0, ROWS)], sem_out).start()
        pltpu.make_async_copy(
            y_sc, y_ref.at[pl.ds(row0, ROWS)], sem_out).wait()
        return carry

    lax.fori_loop(0, ncnt[e], tile, 0)


def _ffn(noff, ncnt, xs, W1, W3, W2):
    grid_spec = pltpu.PrefetchScalarGridSpec(
        num_scalar_prefetch=2,
        grid=(N_EXPERTS,),
        in_specs=[
            pl.BlockSpec(memory_space=pl.ANY),
            pl.BlockSpec((1, D_MODEL, D_FFN), lambda e, no, nc: (e, 0, 0)),
            pl.BlockSpec((1, D_MODEL, D_FFN), lambda e, no, nc: (e, 0, 0)),
            pl.BlockSpec((1, D_FFN, D_MODEL), lambda e, no, nc: (e, 0, 0)),
        ],
        out_specs=pl.BlockSpec(memory_space=pl.ANY),
        scratch_shapes=[
            pltpu.VMEM((ROWS, D_MODEL), jnp.float32),
            pltpu.VMEM((ROWS, D_MODEL), jnp.float32),
            pltpu.SemaphoreType.DMA,
            pltpu.SemaphoreType.DMA,
        ],
    )
    return pl.pallas_call(
        _ffn_body,
        grid_spec=grid_spec,
        out_shape=jax.ShapeDtypeStruct((P, D_MODEL), jnp.float32),
        compiler_params=pltpu.CompilerParams(
            vmem_limit_bytes=100 * 1024 * 1024),
    )(noff, ncnt, xs, W1, W3, W2)


@functools.cache
def _get_combine():
    mesh = plsc.VectorSubcoreMesh(core_axis_name="c", subcore_axis_name="s")

    @functools.partial(
        pl.kernel,
        out_type=jax.ShapeDtypeStruct((T, D_MODEL), jnp.float32),
        mesh=mesh,
        scratch_types=[
            pltpu.VMEM((TPW, D_MODEL), jnp.float32),
            pltpu.VMEM((TPW, D_MODEL), jnp.float32),
            pltpu.VMEM((TPW,), jnp.int32),
            pltpu.VMEM((TPW,), jnp.int32),
            pltpu.VMEM((TPW,), jnp.float32),
            pltpu.VMEM((TPW,), jnp.float32),
            pltpu.SemaphoreType.DMA,
        ],
    )
    def _combine(y_hbm, pos_hbm, w_hbm, out_hbm, b1, b2, idx1, idx2, wv1, wv2, sem):
        wid = lax.axis_index("s") * 2 + lax.axis_index("c")
        base = wid * TPW
        pltpu.sync_copy(pos_hbm.at[wid], idx1)
        pltpu.sync_copy(pos_hbm.at[NW + wid], idx2)
        pltpu.sync_copy(w_hbm.at[wid], wv1)
        pltpu.sync_copy(w_hbm.at[NW + wid], wv2)
        pltpu.async_copy(y_hbm.at[idx1], b1, sem).wait()
        pltpu.async_copy(y_hbm.at[idx2], b2, sem).wait()

        def body(g, carry):
            wvec1 = wv1[pl.ds(g * 16, 16)]
            wvec2 = wv2[pl.ds(g * 16, 16)]
            for lane in range(16):
                t = g * 16 + lane
                ws1 = wvec1[lane]
                ws2 = wvec2[lane]
                for j in range(D_MODEL // 16):
                    s = pl.ds(j * 16, 16)
                    b1[t, s] = ws1 * b1[t, s] + ws2 * b2[t, s]
            return carry

        lax.fori_loop(0, TPW // 16, body, 0)
        pltpu.sync_copy(b1, out_hbm.at[pl.ds(base, TPW)])

    return _combine


def kernel(x, Wg, W1, W2, W3):
    x2 = x.reshape(T, D_MODEL)
    (pos1, pos2, w1n, w2n, arr_e, valid, xblk,
     noff, ncnt, aux) = _router(x2, Wg)
    pos_all = jnp.concatenate(
        [pos1.reshape(NW, TPW), pos2.reshape(NW, TPW)], axis=0)
    w_all = jnp.concatenate(
        [w1n.reshape(NW, TPW), w2n.reshape(NW, TPW)], axis=0)
    xs = _get_dispatch()(x2, pos_all)
    ys = _ffn(noff.reshape(N_EXPERTS), ncnt.reshape(N_EXPERTS),
              xs, W1, W3, W2)
    out2 = _get_combine()(ys, pos_all, w_all)
    return out2.reshape(x.shape), aux.reshape(())


# speedup vs baseline: 1.0969x; 1.0969x over previous
import functools

import jax
import jax.numpy as jnp
from jax import lax
from jax.experimental import pallas as pl
from jax.experimental.pallas import tpu as pltpu
from jax.experimental.pallas import tpu_sc as plsc

D_MODEL = 768
D_FFN = 3072
N_EXPERTS = 8
T = 2048
F_BLK = 768
N_FBLK = D_FFN // F_BLK
ROWS = 256
MAX_TILES = 24
P = MAX_TILES * ROWS
NW = 32
TPW = T // NW


def _router_body(x_ref, wg_ref, pos1_ref, pos2_ref, w1n_ref, w2n_ref,
                 arre_ref, valid_ref, xblk_ref, noff_ref, ncnt_ref, aux_ref):
    xb = x_ref[...]
    logits = jnp.dot(xb, wg_ref[...], preferred_element_type=jnp.float32)
    mu = jnp.mean(logits, axis=1, keepdims=True)
    var = jnp.sum((logits - mu) ** 2, axis=1, keepdims=True) / (N_EXPERTS - 1)
    aux_ref[...] = jnp.mean(var).reshape(1, 1)

    m1 = jnp.max(logits, axis=1, keepdims=True)
    p = jnp.exp(logits - m1)
    probs = p / jnp.sum(p, axis=1, keepdims=True)
    iota8 = lax.broadcasted_iota(jnp.int32, (T, N_EXPERTS), 1)
    w1v = jnp.max(probs, axis=1, keepdims=True)
    e1 = jnp.min(jnp.where(probs == w1v, iota8, N_EXPERTS), axis=1, keepdims=True)
    oh1 = iota8 == e1
    pm = jnp.where(oh1, -1e30, probs)
    w2v = jnp.max(pm, axis=1, keepdims=True)
    e2 = jnp.min(jnp.where(pm == w2v, iota8, N_EXPERTS), axis=1, keepdims=True)
    oh2 = iota8 == e2
    wsum = w1v + w2v
    w1n_ref[...] = w1v / wsum
    w2n_ref[...] = w2v / wsum

    mask = oh1.astype(jnp.float32) + oh2.astype(jnp.float32)
    ir = lax.broadcasted_iota(jnp.int32, (T, T), 0)
    ic = lax.broadcasted_iota(jnp.int32, (T, T), 1)
    lower = (ir > ic).astype(jnp.float32)
    rank = jnp.dot(lower, mask, preferred_element_type=jnp.float32)

    cnt = jnp.sum(mask, axis=0, keepdims=True)
    padded = jnp.ceil(cnt / ROWS) * ROWS
    k8 = lax.broadcasted_iota(jnp.int32, (N_EXPERTS, N_EXPERTS), 0)
    e8 = lax.broadcasted_iota(jnp.int32, (N_EXPERTS, N_EXPERTS), 1)
    lt8 = (k8 < e8).astype(jnp.float32)
    offs = jnp.dot(padded, lt8, preferred_element_type=jnp.float32)
    n_act = jnp.sum(padded) / ROWS

    pos1_ref[...] = (jnp.sum(jnp.where(oh1, rank + offs, 0.0), axis=1,
                             keepdims=True)).astype(jnp.int32)
    pos2_ref[...] = (jnp.sum(jnp.where(oh2, rank + offs, 0.0), axis=1,
                             keepdims=True)).astype(jnp.int32)

    it = lax.broadcasted_iota(jnp.int32, (MAX_TILES, N_EXPERTS), 0).astype(jnp.float32)
    ie = lax.broadcasted_iota(jnp.int32, (MAX_TILES, N_EXPERTS), 1).astype(jnp.float32)
    i_c = jnp.minimum(it, n_act - 1.0)
    start = offs / ROWS
    ntil = padded / ROWS
    ind = ((i_c >= start) & (i_c < start + ntil)).astype(jnp.float32)
    arre_ref[...] = jnp.sum(ie * ind, axis=1, keepdims=True).astype(jnp.int32)
    it1 = lax.broadcasted_iota(jnp.int32, (MAX_TILES, 1), 0).astype(jnp.float32)
    valid_ref[...] = (it1 < n_act).astype(jnp.int32)
    xblk_ref[...] = jnp.minimum(it1, n_act - 1.0).astype(jnp.int32)

    r8 = lax.broadcasted_iota(jnp.int32, (N_EXPERTS, N_EXPERTS), 0)
    c8 = lax.broadcasted_iota(jnp.int32, (N_EXPERTS, N_EXPERTS), 1)
    diag = r8 == c8
    noff_ref[...] = jnp.sum(jnp.where(diag, start, 0.0), axis=1,
                            keepdims=True).astype(jnp.int32)
    ncnt_ref[...] = jnp.sum(jnp.where(diag, ntil, 0.0), axis=1,
                            keepdims=True).astype(jnp.int32)


def _router(x2, Wg):
    outs = pl.pallas_call(
        _router_body,
        out_shape=[
            jax.ShapeDtypeStruct((T, 1), jnp.int32),
            jax.ShapeDtypeStruct((T, 1), jnp.int32),
            jax.ShapeDtypeStruct((T, 1), jnp.float32),
            jax.ShapeDtypeStruct((T, 1), jnp.float32),
            jax.ShapeDtypeStruct((MAX_TILES, 1), jnp.int32),
            jax.ShapeDtypeStruct((MAX_TILES, 1), jnp.int32),
            jax.ShapeDtypeStruct((MAX_TILES, 1), jnp.int32),
            jax.ShapeDtypeStruct((N_EXPERTS, 1), jnp.int32),
            jax.ShapeDtypeStruct((N_EXPERTS, 1), jnp.int32),
            jax.ShapeDtypeStruct((1, 1), jnp.float32),
        ],
    )(x2, Wg)
    return outs


@functools.cache
def _get_dispatch():
    mesh = plsc.VectorSubcoreMesh(core_axis_name="c", subcore_axis_name="s")

    @functools.partial(
        pl.kernel,
        out_type=jax.ShapeDtypeStruct((P, D_MODEL), jnp.float32),
        mesh=mesh,
        scratch_types=[
            pltpu.VMEM((TPW, D_MODEL), jnp.float32),
            pltpu.VMEM((TPW,), jnp.int32),
            pltpu.VMEM((TPW,), jnp.int32),
            pltpu.SemaphoreType.DMA,
        ],
    )
    def _dispatch(x_hbm, pos_hbm, xs_hbm, buf, idx1, idx2, sem):
        wid = lax.axis_index("s") * 2 + lax.axis_index("c")
        base = wid * TPW
        pltpu.sync_copy(x_hbm.at[pl.ds(base, TPW)], buf)
        pltpu.sync_copy(pos_hbm.at[wid], idx1)
        pltpu.sync_copy(pos_hbm.at[NW + wid], idx2)
        pltpu.async_copy(buf, xs_hbm.at[idx1], sem).wait()
        pltpu.async_copy(buf, xs_hbm.at[idx2], sem).wait()

    return _dispatch


def _ffn_body(noff, ncnt, xs_ref, w1_ref, w3_ref, w2_ref, y_ref,
              w1s, w3s, w2s, xb, yb, wsem, xsem, ysem):
    def w_copies(e, slot):
        return (
            pltpu.make_async_copy(w1_ref.at[e], w1s.at[slot], wsem.at[slot]),
            pltpu.make_async_copy(w3_ref.at[e], w3s.at[slot], wsem.at[slot]),
            pltpu.make_async_copy(w2_ref.at[e], w2s.at[slot], wsem.at[slot]),
        )

    def x_copy(row0, xslot):
        return pltpu.make_async_copy(
            xs_ref.at[pl.ds(row0, ROWS)], xb.at[xslot], xsem.at[xslot])

    def y_copy(row0, yslot):
        return pltpu.make_async_copy(
            yb.at[yslot], y_ref.at[pl.ds(row0, ROWS)], ysem.at[yslot])

    for c in w_copies(0, 0):
        c.start()
    for c in w_copies(1, 1):
        c.start()

    for e in range(N_EXPERTS):
        slot = e & 1
        for c in w_copies(e, slot):
            c.wait()
        n = ncnt[e]
        base = noff[e]

        @pl.when(n > 0)
        def _expert():
            x_copy(base * ROWS, 0).start()

            def tile(j, carry):
                xslot = j & 1
                yslot = j & 1
                row0 = (base + j) * ROWS
                x_copy(row0, xslot).wait()

                @pl.when(j + 1 < n)
                def _():
                    x_copy((base + j + 1) * ROWS, xslot ^ 1).start()

                @pl.when(j >= 2)
                def _():
                    y_copy((base + j - 2) * ROWS, yslot).wait()

                xv = xb[xslot]
                a = jnp.dot(xv, w1s[slot], preferred_element_type=jnp.float32)
                b = jnp.dot(xv, w3s[slot], preferred_element_type=jnp.float32)
                h = (a * jax.nn.sigmoid(a)) * b
                yb[yslot] = jnp.dot(h, w2s[slot],
                                    preferred_element_type=jnp.float32)
                y_copy(row0, yslot).start()
                return carry

            lax.fori_loop(0, n, tile, 0)

            @pl.when(n >= 2)
            def _():
                y_copy((base + n - 2) * ROWS, (n - 2) & 1).wait()

            y_copy((base + n - 1) * ROWS, (n - 1) & 1).wait()

        if e + 2 < N_EXPERTS:
            for c in w_copies(e + 2, slot):
                c.start()


def _ffn(noff, ncnt, xs, W1, W3, W2):
    grid_spec = pltpu.PrefetchScalarGridSpec(
        num_scalar_prefetch=2,
        grid=(1,),
        in_specs=[
            pl.BlockSpec(memory_space=pl.ANY),
            pl.BlockSpec(memory_space=pl.ANY),
            pl.BlockSpec(memory_space=pl.ANY),
            pl.BlockSpec(memory_space=pl.ANY),
        ],
        out_specs=pl.BlockSpec(memory_space=pl.ANY),
        scratch_shapes=[
            pltpu.VMEM((2, D_MODEL, D_FFN), jnp.float32),
            pltpu.VMEM((2, D_MODEL, D_FFN), jnp.float32),
            pltpu.VMEM((2, D_FFN, D_MODEL), jnp.float32),
            pltpu.VMEM((2, ROWS, D_MODEL), jnp.float32),
            pltpu.VMEM((2, ROWS, D_MODEL), jnp.float32),
            pltpu.SemaphoreType.DMA((2,)),
            pltpu.SemaphoreType.DMA((2,)),
            pltpu.SemaphoreType.DMA((2,)),
        ],
    )
    return pl.pallas_call(
        _ffn_body,
        grid_spec=grid_spec,
        out_shape=jax.ShapeDtypeStruct((P, D_MODEL), jnp.float32),
        compiler_params=pltpu.CompilerParams(
            vmem_limit_bytes=110 * 1024 * 1024),
    )(noff, ncnt, xs, W1, W3, W2)


@functools.cache
def _get_combine():
    mesh = plsc.VectorSubcoreMesh(core_axis_name="c", subcore_axis_name="s")

    @functools.partial(
        pl.kernel,
        out_type=jax.ShapeDtypeStruct((T, D_MODEL), jnp.float32),
        mesh=mesh,
        scratch_types=[
            pltpu.VMEM((TPW, D_MODEL), jnp.float32),
            pltpu.VMEM((TPW, D_MODEL), jnp.float32),
            pltpu.VMEM((TPW,), jnp.int32),
            pltpu.VMEM((TPW,), jnp.int32),
            pltpu.VMEM((TPW,), jnp.float32),
            pltpu.VMEM((TPW,), jnp.float32),
            pltpu.SemaphoreType.DMA,
        ],
    )
    def _combine(y_hbm, pos_hbm, w_hbm, out_hbm, b1, b2, idx1, idx2, wv1, wv2, sem):
        wid = lax.axis_index("s") * 2 + lax.axis_index("c")
        base = wid * TPW
        pltpu.sync_copy(pos_hbm.at[wid], idx1)
        pltpu.sync_copy(pos_hbm.at[NW + wid], idx2)
        pltpu.sync_copy(w_hbm.at[wid], wv1)
        pltpu.sync_copy(w_hbm.at[NW + wid], wv2)
        pltpu.async_copy(y_hbm.at[idx1], b1, sem).wait()
        pltpu.async_copy(y_hbm.at[idx2], b2, sem).wait()

        def body(g, carry):
            wvec1 = wv1[pl.ds(g * 16, 16)]
            wvec2 = wv2[pl.ds(g * 16, 16)]
            for lane in range(16):
                t = g * 16 + lane
                ws1 = wvec1[lane]
                ws2 = wvec2[lane]
                for j in range(D_MODEL // 16):
                    s = pl.ds(j * 16, 16)
                    b1[t, s] = ws1 * b1[t, s] + ws2 * b2[t, s]
            return carry

        lax.fori_loop(0, TPW // 16, body, 0)
        pltpu.sync_copy(b1, out_hbm.at[pl.ds(base, TPW)])

    return _combine


def kernel(x, Wg, W1, W2, W3):
    x2 = x.reshape(T, D_MODEL)
    (pos1, pos2, w1n, w2n, arr_e, valid, xblk,
     noff, ncnt, aux) = _router(x2, Wg)
    pos_all = jnp.concatenate(
        [pos1.reshape(NW, TPW), pos2.reshape(NW, TPW)], axis=0)
    w_all = jnp.concatenate(
        [w1n.reshape(NW, TPW), w2n.reshape(NW, TPW)], axis=0)
    xs = _get_dispatch()(x2, pos_all)
    ys = _ffn(noff.reshape(N_EXPERTS), ncnt.reshape(N_EXPERTS),
              xs, W1, W3, W2)
    out2 = _get_combine()(ys, pos_all, w_all)
    return out2.reshape(x.shape), aux.reshape(())


# speedup vs baseline: 1.2381x; 1.1288x over previous
import functools

import jax
import jax.numpy as jnp
from jax import lax
from jax.experimental import pallas as pl
from jax.experimental.pallas import tpu as pltpu
from jax.experimental.pallas import tpu_sc as plsc

D_MODEL = 768
D_FFN = 3072
N_EXPERTS = 8
T = 2048
F_BLK = 768
N_FBLK = D_FFN // F_BLK
ROWS = 256
MAX_TILES = 24
P = MAX_TILES * ROWS
NW = 32
TPW = T // NW


def _router_body(x_ref, wg_ref, pos1_ref, pos2_ref, w1n_ref, w2n_ref,
                 arre_ref, valid_ref, xblk_ref, aux_ref):
    xb = x_ref[...]
    logits = jnp.dot(xb, wg_ref[...], preferred_element_type=jnp.float32)
    mu = jnp.mean(logits, axis=1, keepdims=True)
    var = jnp.sum((logits - mu) ** 2, axis=1, keepdims=True) / (N_EXPERTS - 1)
    aux_ref[...] = jnp.mean(var).reshape(1, 1)

    m1 = jnp.max(logits, axis=1, keepdims=True)
    p = jnp.exp(logits - m1)
    probs = p / jnp.sum(p, axis=1, keepdims=True)
    iota8 = lax.broadcasted_iota(jnp.int32, (T, N_EXPERTS), 1)
    w1v = jnp.max(probs, axis=1, keepdims=True)
    e1 = jnp.min(jnp.where(probs == w1v, iota8, N_EXPERTS), axis=1, keepdims=True)
    oh1 = iota8 == e1
    pm = jnp.where(oh1, -1e30, probs)
    w2v = jnp.max(pm, axis=1, keepdims=True)
    e2 = jnp.min(jnp.where(pm == w2v, iota8, N_EXPERTS), axis=1, keepdims=True)
    oh2 = iota8 == e2
    wsum = w1v + w2v
    w1n_ref[...] = w1v / wsum
    w2n_ref[...] = w2v / wsum

    mask = oh1.astype(jnp.float32) + oh2.astype(jnp.float32)
    ir = lax.broadcasted_iota(jnp.int32, (T, T), 0)
    ic = lax.broadcasted_iota(jnp.int32, (T, T), 1)
    lower = (ir > ic).astype(jnp.float32)
    rank = jnp.dot(lower, mask, preferred_element_type=jnp.float32)

    cnt = jnp.sum(mask, axis=0, keepdims=True)
    padded = jnp.ceil(cnt / ROWS) * ROWS
    k8 = lax.broadcasted_iota(jnp.int32, (N_EXPERTS, N_EXPERTS), 0)
    e8 = lax.broadcasted_iota(jnp.int32, (N_EXPERTS, N_EXPERTS), 1)
    lt8 = (k8 < e8).astype(jnp.float32)
    offs = jnp.dot(padded, lt8, preferred_element_type=jnp.float32)
    n_act = jnp.sum(padded) / ROWS

    pos1_ref[...] = (jnp.sum(jnp.where(oh1, rank + offs, 0.0), axis=1,
                             keepdims=True)).astype(jnp.int32)
    pos2_ref[...] = (jnp.sum(jnp.where(oh2, rank + offs, 0.0), axis=1,
                             keepdims=True)).astype(jnp.int32)

    it = lax.broadcasted_iota(jnp.int32, (MAX_TILES, N_EXPERTS), 0).astype(jnp.float32)
    ie = lax.broadcasted_iota(jnp.int32, (MAX_TILES, N_EXPERTS), 1).astype(jnp.float32)
    i_c = jnp.minimum(it, n_act - 1.0)
    start = offs / ROWS
    ntil = padded / ROWS
    ind = ((i_c >= start) & (i_c < start + ntil)).astype(jnp.float32)
    arre_ref[...] = jnp.sum(ie * ind, axis=1, keepdims=True).astype(jnp.int32)
    it1 = lax.broadcasted_iota(jnp.int32, (MAX_TILES, 1), 0).astype(jnp.float32)
    valid_ref[...] = (it1 < n_act).astype(jnp.int32)
    xblk_ref[...] = jnp.minimum(it1, n_act - 1.0).astype(jnp.int32)


def _router(x2, Wg):
    outs = pl.pallas_call(
        _router_body,
        out_shape=[
            jax.ShapeDtypeStruct((T, 1), jnp.int32),
            jax.ShapeDtypeStruct((T, 1), jnp.int32),
            jax.ShapeDtypeStruct((T, 1), jnp.float32),
            jax.ShapeDtypeStruct((T, 1), jnp.float32),
            jax.ShapeDtypeStruct((MAX_TILES, 1), jnp.int32),
            jax.ShapeDtypeStruct((MAX_TILES, 1), jnp.int32),
            jax.ShapeDtypeStruct((MAX_TILES, 1), jnp.int32),
            jax.ShapeDtypeStruct((1, 1), jnp.float32),
        ],
    )(x2, Wg)
    return outs


@functools.cache
def _get_dispatch():
    mesh = plsc.VectorSubcoreMesh(core_axis_name="c", subcore_axis_name="s")

    @functools.partial(
        pl.kernel,
        out_type=jax.ShapeDtypeStruct((P, D_MODEL), jnp.float32),
        mesh=mesh,
        scratch_types=[
            pltpu.VMEM((TPW, D_MODEL), jnp.float32),
            pltpu.VMEM((TPW,), jnp.int32),
            pltpu.VMEM((TPW,), jnp.int32),
            pltpu.SemaphoreType.DMA,
        ],
    )
    def _dispatch(x_hbm, pos_hbm, xs_hbm, buf, idx1, idx2, sem):
        wid = lax.axis_index("s") * 2 + lax.axis_index("c")
        base = wid * TPW
        pltpu.sync_copy(x_hbm.at[pl.ds(base, TPW)], buf)
        pltpu.sync_copy(pos_hbm.at[wid], idx1)
        pltpu.sync_copy(pos_hbm.at[NW + wid], idx2)
        pltpu.async_copy(buf, xs_hbm.at[idx1], sem).wait()
        pltpu.async_copy(buf, xs_hbm.at[idx2], sem).wait()

    return _dispatch


def _ffn_body(se, sv, sx, x_ref, w1_ref, w3_ref, w2_ref, y_ref):
    i = pl.program_id(0)

    @pl.when(sv[i] == 1)
    def _():
        xb = x_ref[...]
        a = jnp.dot(xb, w1_ref[0], preferred_element_type=jnp.float32)
        b = jnp.dot(xb, w3_ref[0], preferred_element_type=jnp.float32)
        h = (a * jax.nn.sigmoid(a)) * b
        y_ref[...] = jnp.dot(h, w2_ref[0], preferred_element_type=jnp.float32)


def _ffn(arr_e, valid, xblk, xs, W1, W3, W2):
    grid_spec = pltpu.PrefetchScalarGridSpec(
        num_scalar_prefetch=3,
        grid=(MAX_TILES,),
        in_specs=[
            pl.BlockSpec((ROWS, D_MODEL), lambda i, se, sv, sx: (sx[i], 0)),
            pl.BlockSpec((1, D_MODEL, D_FFN), lambda i, se, sv, sx: (se[i], 0, 0)),
            pl.BlockSpec((1, D_MODEL, D_FFN), lambda i, se, sv, sx: (se[i], 0, 0)),
            pl.BlockSpec((1, D_FFN, D_MODEL), lambda i, se, sv, sx: (se[i], 0, 0)),
        ],
        out_specs=pl.BlockSpec((ROWS, D_MODEL), lambda i, se, sv, sx: (sx[i], 0)),
    )
    return pl.pallas_call(
        _ffn_body,
        grid_spec=grid_spec,
        out_shape=jax.ShapeDtypeStruct((P, D_MODEL), jnp.float32),
        compiler_params=pltpu.CompilerParams(
            vmem_limit_bytes=100 * 1024 * 1024),
    )(arr_e, valid, xblk, xs, W1, W3, W2)


HT = TPW // 2


@functools.cache
def _get_combine():
    mesh = plsc.VectorSubcoreMesh(core_axis_name="c", subcore_axis_name="s")

    @functools.partial(
        pl.kernel,
        out_type=jax.ShapeDtypeStruct((T, D_MODEL), jnp.float32),
        mesh=mesh,
        scratch_types=[
            pltpu.VMEM((HT, D_MODEL), jnp.float32),
            pltpu.VMEM((HT, D_MODEL), jnp.float32),
            pltpu.VMEM((HT, D_MODEL), jnp.float32),
            pltpu.VMEM((HT, D_MODEL), jnp.float32),
            pltpu.VMEM((HT,), jnp.int32),
            pltpu.VMEM((HT,), jnp.int32),
            pltpu.VMEM((HT,), jnp.int32),
            pltpu.VMEM((HT,), jnp.int32),
            pltpu.VMEM((HT,), jnp.float32),
            pltpu.VMEM((HT,), jnp.float32),
            pltpu.VMEM((HT,), jnp.float32),
            pltpu.VMEM((HT,), jnp.float32),
            pltpu.SemaphoreType.DMA,
            pltpu.SemaphoreType.DMA,
            pltpu.SemaphoreType.DMA,
            pltpu.SemaphoreType.DMA,
            pltpu.SemaphoreType.DMA,
            pltpu.SemaphoreType.DMA,
        ],
    )
    def _combine(y_hbm, pos_hbm, w_hbm, out_hbm,
                 g1a, g2a, g1b, g2b, i1a, i1b, i2a, i2b,
                 v1a, v1b, v2a, v2b, s1a, s2a, s1b, s2b, soa, sob):
        wid = lax.axis_index("s") * 2 + lax.axis_index("c")
        base = wid * TPW
        pltpu.sync_copy(pos_hbm.at[0 * NW + wid], i1a)
        pltpu.sync_copy(pos_hbm.at[1 * NW + wid], i1b)
        pltpu.sync_copy(pos_hbm.at[2 * NW + wid], i2a)
        pltpu.sync_copy(pos_hbm.at[3 * NW + wid], i2b)
        pltpu.sync_copy(w_hbm.at[0 * NW + wid], v1a)
        pltpu.sync_copy(w_hbm.at[1 * NW + wid], v1b)
        pltpu.sync_copy(w_hbm.at[2 * NW + wid], v2a)
        pltpu.sync_copy(w_hbm.at[3 * NW + wid], v2b)
        h1a = pltpu.async_copy(y_hbm.at[i1a], g1a, s1a)
        h2a = pltpu.async_copy(y_hbm.at[i2a], g2a, s2a)
        h1b = pltpu.async_copy(y_hbm.at[i1b], g1b, s1b)
        h2b = pltpu.async_copy(y_hbm.at[i2b], g2b, s2b)

        def chunk(b1, b2, wv1, wv2):
            def body(g, carry):
                wvec1 = wv1[pl.ds(g * 16, 16)]
                wvec2 = wv2[pl.ds(g * 16, 16)]
                for lane in range(16):
                    t = g * 16 + lane
                    ws1 = wvec1[lane]
                    ws2 = wvec2[lane]
                    for j in range(D_MODEL // 16):
                        s = pl.ds(j * 16, 16)
                        b1[t, s] = ws1 * b1[t, s] + ws2 * b2[t, s]
                return carry

            lax.fori_loop(0, HT // 16, body, 0)

        h1a.wait()
        h2a.wait()
        chunk(g1a, g2a, v1a, v2a)
        wba = pltpu.async_copy(g1a, out_hbm.at[pl.ds(base, HT)], soa)
        h1b.wait()
        h2b.wait()
        chunk(g1b, g2b, v1b, v2b)
        wbb = pltpu.async_copy(g1b, out_hbm.at[pl.ds(base + HT, HT)], sob)
        wba.wait()
        wbb.wait()

    return _combine


def kernel(x, Wg, W1, W2, W3):
    x2 = x.reshape(T, D_MODEL)
    pos1, pos2, w1n, w2n, arr_e, valid, xblk, aux = _router(x2, Wg)
    pos_all = jnp.concatenate(
        [pos1.reshape(NW, TPW), pos2.reshape(NW, TPW)], axis=0)
    w_all = jnp.concatenate(
        [w1n.reshape(NW, TPW), w2n.reshape(NW, TPW)], axis=0)
    xs = _get_dispatch()(x2, pos_all)
    ys = _ffn(arr_e.reshape(MAX_TILES), valid.reshape(MAX_TILES),
              xblk.reshape(MAX_TILES), xs, W1, W3, W2)
    p1c = pos1.reshape(NW, 2, HT)
    p2c = pos2.reshape(NW, 2, HT)
    w1c = w1n.reshape(NW, 2, HT)
    w2c = w2n.reshape(NW, 2, HT)
    pos_c = jnp.concatenate(
        [p1c[:, 0], p1c[:, 1], p2c[:, 0], p2c[:, 1]], axis=0)
    w_c = jnp.concatenate(
        [w1c[:, 0], w1c[:, 1], w2c[:, 0], w2c[:, 1]], axis=0)
    out2 = _get_combine()(ys, pos_c, w_c)
    return out2.reshape(x.shape), aux.reshape(())


# speedup vs baseline: 1.2560x; 1.0145x over previous
import functools

import jax
import jax.numpy as jnp
from jax import lax
from jax.experimental import pallas as pl
from jax.experimental.pallas import tpu as pltpu
from jax.experimental.pallas import tpu_sc as plsc

D_MODEL = 768
D_FFN = 3072
N_EXPERTS = 8
T = 2048
F_BLK = 768
N_FBLK = D_FFN // F_BLK
ROWS = 256
MAX_TILES = 24
P = MAX_TILES * ROWS
NW = 32
TPW = T // NW


def _router_body(x_ref, wg_ref, pos1_ref, pos2_ref, w1n_ref, w2n_ref,
                 arre_ref, valid_ref, xblk_ref, aux_ref):
    xb = x_ref[...]
    logits = jnp.dot(xb, wg_ref[...], preferred_element_type=jnp.float32)
    mu = jnp.mean(logits, axis=1, keepdims=True)
    var = jnp.sum((logits - mu) ** 2, axis=1, keepdims=True) / (N_EXPERTS - 1)
    aux_ref[...] = jnp.mean(var).reshape(1, 1)

    m1 = jnp.max(logits, axis=1, keepdims=True)
    p = jnp.exp(logits - m1)
    probs = p / jnp.sum(p, axis=1, keepdims=True)
    iota8 = lax.broadcasted_iota(jnp.int32, (T, N_EXPERTS), 1)
    w1v = jnp.max(probs, axis=1, keepdims=True)
    e1 = jnp.min(jnp.where(probs == w1v, iota8, N_EXPERTS), axis=1, keepdims=True)
    oh1 = iota8 == e1
    pm = jnp.where(oh1, -1e30, probs)
    w2v = jnp.max(pm, axis=1, keepdims=True)
    e2 = jnp.min(jnp.where(pm == w2v, iota8, N_EXPERTS), axis=1, keepdims=True)
    oh2 = iota8 == e2
    wsum = w1v + w2v
    w1n_ref[...] = w1v / wsum
    w2n_ref[...] = w2v / wsum

    mask = oh1.astype(jnp.float32) + oh2.astype(jnp.float32)
    ir = lax.broadcasted_iota(jnp.int32, (T, T), 0)
    ic = lax.broadcasted_iota(jnp.int32, (T, T), 1)
    lower = (ir > ic).astype(jnp.float32)
    rank = jnp.dot(lower, mask, preferred_element_type=jnp.float32)

    cnt = jnp.sum(mask, axis=0, keepdims=True)
    padded = jnp.ceil(cnt / ROWS) * ROWS
    k8 = lax.broadcasted_iota(jnp.int32, (N_EXPERTS, N_EXPERTS), 0)
    e8 = lax.broadcasted_iota(jnp.int32, (N_EXPERTS, N_EXPERTS), 1)
    lt8 = (k8 < e8).astype(jnp.float32)
    offs = jnp.dot(padded, lt8, preferred_element_type=jnp.float32)
    n_act = jnp.sum(padded) / ROWS

    pos1_ref[...] = (jnp.sum(jnp.where(oh1, rank + offs, 0.0), axis=1,
                             keepdims=True)).astype(jnp.int32)
    pos2_ref[...] = (jnp.sum(jnp.where(oh2, rank + offs, 0.0), axis=1,
                             keepdims=True)).astype(jnp.int32)

    it = lax.broadcasted_iota(jnp.int32, (MAX_TILES, N_EXPERTS), 0).astype(jnp.float32)
    ie = lax.broadcasted_iota(jnp.int32, (MAX_TILES, N_EXPERTS), 1).astype(jnp.float32)
    i_c = jnp.minimum(it, n_act - 1.0)
    start = offs / ROWS
    ntil = padded / ROWS
    ind = ((i_c >= start) & (i_c < start + ntil)).astype(jnp.float32)
    arre_ref[...] = jnp.sum(ie * ind, axis=1, keepdims=True).astype(jnp.int32)
    it1 = lax.broadcasted_iota(jnp.int32, (MAX_TILES, 1), 0).astype(jnp.float32)
    valid_ref[...] = (it1 < n_act).astype(jnp.int32)
    xblk_ref[...] = jnp.minimum(it1, n_act - 1.0).astype(jnp.int32)


def _router(x2, Wg):
    outs = pl.pallas_call(
        _router_body,
        out_shape=[
            jax.ShapeDtypeStruct((T, 1), jnp.int32),
            jax.ShapeDtypeStruct((T, 1), jnp.int32),
            jax.ShapeDtypeStruct((T, 1), jnp.float32),
            jax.ShapeDtypeStruct((T, 1), jnp.float32),
            jax.ShapeDtypeStruct((MAX_TILES, 1), jnp.int32),
            jax.ShapeDtypeStruct((MAX_TILES, 1), jnp.int32),
            jax.ShapeDtypeStruct((MAX_TILES, 1), jnp.int32),
            jax.ShapeDtypeStruct((1, 1), jnp.float32),
        ],
    )(x2, Wg)
    return outs


@functools.cache
def _get_dispatch():
    mesh = plsc.VectorSubcoreMesh(core_axis_name="c", subcore_axis_name="s")

    @functools.partial(
        pl.kernel,
        out_type=jax.ShapeDtypeStruct((P, D_MODEL), jnp.float32),
        mesh=mesh,
        scratch_types=[
            pltpu.VMEM((TPW, D_MODEL), jnp.float32),
            pltpu.VMEM((TPW,), jnp.int32),
            pltpu.VMEM((TPW,), jnp.int32),
            pltpu.SemaphoreType.DMA,
        ],
    )
    def _dispatch(x_hbm, pos_hbm, xs_hbm, buf, idx1, idx2, sem):
        wid = lax.axis_index("s") * 2 + lax.axis_index("c")
        base = wid * TPW
        pltpu.sync_copy(x_hbm.at[pl.ds(base, TPW)], buf)
        pltpu.sync_copy(pos_hbm.at[wid], idx1)
        pltpu.sync_copy(pos_hbm.at[NW + wid], idx2)
        pltpu.async_copy(buf, xs_hbm.at[idx1], sem).wait()
        pltpu.async_copy(buf, xs_hbm.at[idx2], sem).wait()

    return _dispatch


def _ffn_body(se, sv, sx, x_ref, w1_ref, w3_ref, w2_ref, y_ref):
    i = pl.program_id(0)

    @pl.when(sv[i] == 1)
    def _():
        xb = x_ref[...]
        a = jnp.dot(xb, w1_ref[0], preferred_element_type=jnp.float32,
                    precision=lax.Precision.DEFAULT)
        b = jnp.dot(xb, w3_ref[0], preferred_element_type=jnp.float32,
                    precision=lax.Precision.DEFAULT)
        h = (a * jax.nn.sigmoid(a)) * b
        y_ref[...] = jnp.dot(h, w2_ref[0], preferred_element_type=jnp.float32,
                             precision=lax.Precision.DEFAULT)


def _ffn(arr_e, valid, xblk, xs, W1, W3, W2):
    grid_spec = pltpu.PrefetchScalarGridSpec(
        num_scalar_prefetch=3,
        grid=(MAX_TILES,),
        in_specs=[
            pl.BlockSpec((ROWS, D_MODEL), lambda i, se, sv, sx: (sx[i], 0)),
            pl.BlockSpec((1, D_MODEL, D_FFN), lambda i, se, sv, sx: (se[i], 0, 0)),
            pl.BlockSpec((1, D_MODEL, D_FFN), lambda i, se, sv, sx: (se[i], 0, 0)),
            pl.BlockSpec((1, D_FFN, D_MODEL), lambda i, se, sv, sx: (se[i], 0, 0)),
        ],
        out_specs=pl.BlockSpec((ROWS, D_MODEL), lambda i, se, sv, sx: (sx[i], 0)),
    )
    return pl.pallas_call(
        _ffn_body,
        grid_spec=grid_spec,
        out_shape=jax.ShapeDtypeStruct((P, D_MODEL), jnp.float32),
        compiler_params=pltpu.CompilerParams(
            vmem_limit_bytes=100 * 1024 * 1024),
    )(arr_e, valid, xblk, xs, W1, W3, W2)


@functools.cache
def _get_combine():
    mesh = plsc.VectorSubcoreMesh(core_axis_name="c", subcore_axis_name="s")

    @functools.partial(
        pl.kernel,
        out_type=jax.ShapeDtypeStruct((T, D_MODEL), jnp.float32),
        mesh=mesh,
        scratch_types=[
            pltpu.VMEM((TPW, D_MODEL), jnp.float32),
            pltpu.VMEM((TPW, D_MODEL), jnp.float32),
            pltpu.VMEM((TPW,), jnp.int32),
            pltpu.VMEM((TPW,), jnp.int32),
            pltpu.VMEM((TPW,), jnp.float32),
            pltpu.VMEM((TPW,), jnp.float32),
            pltpu.SemaphoreType.DMA,
        ],
    )
    def _combine(y_hbm, pos_hbm, w_hbm, out_hbm, b1, b2, idx1, idx2, wv1, wv2, sem):
        wid = lax.axis_index("s") * 2 + lax.axis_index("c")
        base = wid * TPW
        pltpu.sync_copy(pos_hbm.at[wid], idx1)
        pltpu.sync_copy(pos_hbm.at[NW + wid], idx2)
        pltpu.sync_copy(w_hbm.at[wid], wv1)
        pltpu.sync_copy(w_hbm.at[NW + wid], wv2)
        pltpu.async_copy(y_hbm.at[idx1], b1, sem).wait()
        pltpu.async_copy(y_hbm.at[idx2], b2, sem).wait()

        def body(g, carry):
            wvec1 = wv1[pl.ds(g * 16, 16)]
            wvec2 = wv2[pl.ds(g * 16, 16)]
            for lane in range(16):
                t = g * 16 + lane
                ws1 = wvec1[lane]
                ws2 = wvec2[lane]
                for j in range(D_MODEL // 16):
                    s = pl.ds(j * 16, 16)
                    b1[t, s] = ws1 * b1[t, s] + ws2 * b2[t, s]
            return carry

        lax.fori_loop(0, TPW // 16, body, 0)
        pltpu.sync_copy(b1, out_hbm.at[pl.ds(base, TPW)])

    return _combine


def kernel(x, Wg, W1, W2, W3):
    x2 = x.reshape(T, D_MODEL)
    pos1, pos2, w1n, w2n, arr_e, valid, xblk, aux = _router(x2, Wg)
    pos_all = jnp.concatenate(
        [pos1.reshape(NW, TPW), pos2.reshape(NW, TPW)], axis=0)
    w_all = jnp.concatenate(
        [w1n.reshape(NW, TPW), w2n.reshape(NW, TPW)], axis=0)
    xs = _get_dispatch()(x2, pos_all)
    ys = _ffn(arr_e.reshape(MAX_TILES), valid.reshape(MAX_TILES),
              xblk.reshape(MAX_TILES), xs, W1, W3, W2)
    out2 = _get_combine()(ys, pos_all, w_all)
    return out2.reshape(x.shape), aux.reshape(())


# speedup vs baseline: 1.2567x; 1.0006x over previous
import functools

import jax
import jax.numpy as jnp
from jax import lax
from jax.experimental import pallas as pl
from jax.experimental.pallas import tpu as pltpu
from jax.experimental.pallas import tpu_sc as plsc

D_MODEL = 768
D_FFN = 3072
N_EXPERTS = 8
T = 2048
F_BLK = 768
N_FBLK = D_FFN // F_BLK
ROWS = 256
MAX_TILES = 24
P = MAX_TILES * ROWS
NW = 32
TPW = T // NW


def _router_body(x_ref, wg_ref, pos1_ref, pos2_ref, w1n_ref, w2n_ref,
                 arre_ref, valid_ref, xblk_ref, aux_ref):
    xb = x_ref[...]
    logits = jnp.dot(xb, wg_ref[...], preferred_element_type=jnp.float32)
    mu = jnp.mean(logits, axis=1, keepdims=True)
    var = jnp.sum((logits - mu) ** 2, axis=1, keepdims=True) / (N_EXPERTS - 1)
    aux_ref[...] = jnp.mean(var).reshape(1, 1)

    m1 = jnp.max(logits, axis=1, keepdims=True)
    p = jnp.exp(logits - m1)
    probs = p / jnp.sum(p, axis=1, keepdims=True)
    iota8 = lax.broadcasted_iota(jnp.int32, (T, N_EXPERTS), 1)
    w1v = jnp.max(probs, axis=1, keepdims=True)
    e1 = jnp.min(jnp.where(probs == w1v, iota8, N_EXPERTS), axis=1, keepdims=True)
    oh1 = iota8 == e1
    pm = jnp.where(oh1, -1e30, probs)
    w2v = jnp.max(pm, axis=1, keepdims=True)
    e2 = jnp.min(jnp.where(pm == w2v, iota8, N_EXPERTS), axis=1, keepdims=True)
    oh2 = iota8 == e2
    wsum = w1v + w2v
    w1n_ref[...] = w1v / wsum
    w2n_ref[...] = w2v / wsum

    mask = oh1.astype(jnp.float32) + oh2.astype(jnp.float32)
    ir = lax.broadcasted_iota(jnp.int32, (T, T), 0)
    ic = lax.broadcasted_iota(jnp.int32, (T, T), 1)
    lower = (ir > ic).astype(jnp.float32)
    rank = jnp.dot(lower, mask, preferred_element_type=jnp.float32)

    cnt = jnp.sum(mask, axis=0, keepdims=True)
    padded = jnp.ceil(cnt / ROWS) * ROWS
    k8 = lax.broadcasted_iota(jnp.int32, (N_EXPERTS, N_EXPERTS), 0)
    e8 = lax.broadcasted_iota(jnp.int32, (N_EXPERTS, N_EXPERTS), 1)
    lt8 = (k8 < e8).astype(jnp.float32)
    offs = jnp.dot(padded, lt8, preferred_element_type=jnp.float32)
    n_act = jnp.sum(padded) / ROWS

    pos1_ref[...] = (jnp.sum(jnp.where(oh1, rank + offs, 0.0), axis=1,
                             keepdims=True)).astype(jnp.int32)
    pos2_ref[...] = (jnp.sum(jnp.where(oh2, rank + offs, 0.0), axis=1,
                             keepdims=True)).astype(jnp.int32)

    it = lax.broadcasted_iota(jnp.int32, (MAX_TILES, N_EXPERTS), 0).astype(jnp.float32)
    ie = lax.broadcasted_iota(jnp.int32, (MAX_TILES, N_EXPERTS), 1).astype(jnp.float32)
    i_c = jnp.minimum(it, n_act - 1.0)
    start = offs / ROWS
    ntil = padded / ROWS
    ind = ((i_c >= start) & (i_c < start + ntil)).astype(jnp.float32)
    arre_ref[...] = jnp.sum(ie * ind, axis=1, keepdims=True).astype(jnp.int32)
    it1 = lax.broadcasted_iota(jnp.int32, (MAX_TILES, 1), 0).astype(jnp.float32)
    valid_ref[...] = (it1 < n_act).astype(jnp.int32)
    xblk_ref[...] = jnp.minimum(it1, n_act - 1.0).astype(jnp.int32)


def _router(x2, Wg):
    outs = pl.pallas_call(
        _router_body,
        out_shape=[
            jax.ShapeDtypeStruct((T, 1), jnp.int32),
            jax.ShapeDtypeStruct((T, 1), jnp.int32),
            jax.ShapeDtypeStruct((T, 1), jnp.float32),
            jax.ShapeDtypeStruct((T, 1), jnp.float32),
            jax.ShapeDtypeStruct((MAX_TILES, 1), jnp.int32),
            jax.ShapeDtypeStruct((MAX_TILES, 1), jnp.int32),
            jax.ShapeDtypeStruct((MAX_TILES, 1), jnp.int32),
            jax.ShapeDtypeStruct((1, 1), jnp.float32),
        ],
    )(x2, Wg)
    return outs


@functools.cache
def _get_dispatch():
    mesh = plsc.VectorSubcoreMesh(core_axis_name="c", subcore_axis_name="s")

    @functools.partial(
        pl.kernel,
        out_type=jax.ShapeDtypeStruct((P, D_MODEL), jnp.float32),
        mesh=mesh,
        scratch_types=[
            pltpu.VMEM((TPW, D_MODEL), jnp.float32),
            pltpu.VMEM((TPW,), jnp.int32),
            pltpu.VMEM((TPW,), jnp.int32),
            pltpu.SemaphoreType.DMA,
        ],
    )
    def _dispatch(x_hbm, pos_hbm, xs_hbm, buf, idx1, idx2, sem):
        wid = lax.axis_index("s") * 2 + lax.axis_index("c")
        base = wid * TPW
        pltpu.sync_copy(x_hbm.at[pl.ds(base, TPW)], buf)
        pltpu.sync_copy(pos_hbm.at[wid], idx1)
        pltpu.sync_copy(pos_hbm.at[NW + wid], idx2)
        pltpu.async_copy(buf, xs_hbm.at[idx1], sem).wait()
        pltpu.async_copy(buf, xs_hbm.at[idx2], sem).wait()

    return _dispatch


def _ffn_body(se, sv, sx, x_ref, w1_ref, w3_ref, w2_ref, y_ref):
    i = pl.program_id(0)

    @pl.when(sv[i] == 1)
    def _():
        xb = x_ref[...]
        a = jnp.dot(xb, w1_ref[0], preferred_element_type=jnp.float32)
        b = jnp.dot(xb, w3_ref[0], preferred_element_type=jnp.float32)
        h = (a * jax.nn.sigmoid(a)) * b
        y_ref[...] = jnp.dot(h, w2_ref[0], preferred_element_type=jnp.float32)


def _ffn(arr_e, valid, xblk, xs, W1, W3, W2):
    grid_spec = pltpu.PrefetchScalarGridSpec(
        num_scalar_prefetch=3,
        grid=(MAX_TILES,),
        in_specs=[
            pl.BlockSpec((ROWS, D_MODEL), lambda i, se, sv, sx: (sx[i], 0)),
            pl.BlockSpec((1, D_MODEL, D_FFN), lambda i, se, sv, sx: (se[i], 0, 0)),
            pl.BlockSpec((1, D_MODEL, D_FFN), lambda i, se, sv, sx: (se[i], 0, 0)),
            pl.BlockSpec((1, D_FFN, D_MODEL), lambda i, se, sv, sx: (se[i], 0, 0)),
        ],
        out_specs=pl.BlockSpec((ROWS, D_MODEL), lambda i, se, sv, sx: (sx[i], 0)),
    )
    return pl.pallas_call(
        _ffn_body,
        grid_spec=grid_spec,
        out_shape=jax.ShapeDtypeStruct((P, D_MODEL), jnp.float32),
        compiler_params=pltpu.CompilerParams(
            vmem_limit_bytes=100 * 1024 * 1024),
    )(arr_e, valid, xblk, xs, W1, W3, W2)


@functools.cache
def _get_combine():
    mesh = plsc.VectorSubcoreMesh(core_axis_name="c", subcore_axis_name="s")

    @functools.partial(
        pl.kernel,
        out_type=jax.ShapeDtypeStruct((T, D_MODEL), jnp.float32),
        mesh=mesh,
        scratch_types=[
            pltpu.VMEM((TPW, D_MODEL), jnp.float32),
            pltpu.VMEM((TPW, D_MODEL), jnp.float32),
            pltpu.VMEM((TPW,), jnp.int32),
            pltpu.VMEM((TPW,), jnp.int32),
            pltpu.VMEM((TPW,), jnp.float32),
            pltpu.VMEM((TPW,), jnp.float32),
            pltpu.SemaphoreType.DMA,
        ],
    )
    def _combine(y_hbm, pos_hbm, w_hbm, out_hbm, b1, b2, idx1, idx2, wv1, wv2, sem):
        wid = lax.axis_index("s") * 2 + lax.axis_index("c")
        base = wid * TPW
        pltpu.sync_copy(pos_hbm.at[wid], idx1)
        pltpu.sync_copy(pos_hbm.at[NW + wid], idx2)
        pltpu.sync_copy(w_hbm.at[wid], wv1)
        pltpu.sync_copy(w_hbm.at[NW + wid], wv2)
        pltpu.async_copy(y_hbm.at[idx1], b1, sem).wait()
        pltpu.async_copy(y_hbm.at[idx2], b2, sem).wait()

        def body(g, carry):
            wvec1 = wv1[pl.ds(g * 16, 16)]
            wvec2 = wv2[pl.ds(g * 16, 16)]
            for lane in range(16):
                t = g * 16 + lane
                ws1 = wvec1[lane]
                ws2 = wvec2[lane]
                for j in range(D_MODEL // 16):
                    s = pl.ds(j * 16, 16)
                    b1[t, s] = ws1 * b1[t, s] + ws2 * b2[t, s]
            return carry

        lax.fori_loop(0, TPW // 16, body, 0)
        pltpu.sync_copy(b1, out_hbm.at[pl.ds(base, TPW)])

    return _combine


def kernel(x, Wg, W1, W2, W3):
    x2 = x.reshape(T, D_MODEL)
    pos1, pos2, w1n, w2n, arr_e, valid, xblk, aux = _router(x2, Wg)
    pos_all = jnp.concatenate(
        [pos1.reshape(NW, TPW), pos2.reshape(NW, TPW)], axis=0)
    w_all = jnp.concatenate(
        [w1n.reshape(NW, TPW), w2n.reshape(NW, TPW)], axis=0)
    xs = _get_dispatch()(x2, pos_all)
    ys = _ffn(arr_e.reshape(MAX_TILES), valid.reshape(MAX_TILES),
              xblk.reshape(MAX_TILES), xs, W1, W3, W2)
    out2 = _get_combine()(ys, pos_all, w_all)
    return out2.reshape(x.shape), aux.reshape(())
